# baseline (speedup 1.0000x reference)
import functools

import jax
import jax.numpy as jnp
from jax import lax
from jax.experimental import pallas as pl
from jax.experimental.pallas import tpu as pltpu
from jax.experimental.pallas import tpu_sc as plsc

NC = 2
NS = 16
NW = NC * NS
CHUNK = 128


def kernel(indices, embeddings):
    B, S = indices.shape
    V, D = embeddings.shape
    total = B * S
    per_w = total // NW
    n_chunks = per_w // CHUNK
    assert per_w * NW == total and n_chunks * CHUNK == per_w

    idx3 = indices.reshape(NW, n_chunks, CHUNK)
    mesh = plsc.VectorSubcoreMesh(core_axis_name="c", subcore_axis_name="s")

    @functools.partial(
        pl.kernel,
        out_type=jax.ShapeDtypeStruct((total, D), jnp.float32),
        mesh=mesh,
        compiler_params=pltpu.CompilerParams(use_tc_tiling_on_sc=False),
        scratch_types=[
            pltpu.VMEM((n_chunks, CHUNK), jnp.int32),
            pltpu.VMEM((CHUNK, D), jnp.float32),
            pltpu.SemaphoreType.DMA,
        ],
    )
    def gather_kernel(idx_hbm, tab_hbm, out_hbm, idx_v, rows, sem):
        wid = lax.axis_index("s") * NC + lax.axis_index("c")
        base = wid * per_w
        pltpu.sync_copy(idx_hbm.at[wid], idx_v)

        def body(g, carry):
            pltpu.async_copy(tab_hbm.at[idx_v.at[g]], rows, sem).wait()
            pltpu.sync_copy(rows, out_hbm.at[pl.ds(base + g * CHUNK, CHUNK)])
            return carry

        lax.fori_loop(0, n_chunks, body, 0)

    out = gather_kernel(idx3, embeddings)
    return out.reshape(B, S, D)


# speedup vs baseline: 1.1194x; 1.1194x over previous
import functools

import jax
import jax.numpy as jnp
from jax import lax
from jax.experimental import pallas as pl
from jax.experimental.pallas import tpu as pltpu
from jax.experimental.pallas import tpu_sc as plsc

NC = 2
NS = 16
NW = NC * NS
CHUNK = 128
NBUF = 8


def kernel(indices, embeddings):
    B, S = indices.shape
    V, D = embeddings.shape
    total = B * S
    per_w = total // NW
    n_chunks = per_w // CHUNK
    assert per_w * NW == total and n_chunks * CHUNK == per_w

    idx3 = indices.reshape(NW, n_chunks, CHUNK)
    mesh = plsc.VectorSubcoreMesh(core_axis_name="c", subcore_axis_name="s")

    @functools.partial(
        pl.kernel,
        out_type=jax.ShapeDtypeStruct((total, D), jnp.float32),
        mesh=mesh,
        compiler_params=pltpu.CompilerParams(use_tc_tiling_on_sc=False),
        scratch_types=[
            pltpu.VMEM((n_chunks, CHUNK), jnp.int32),
            pltpu.VMEM((NBUF, CHUNK, D), jnp.float32),
            pltpu.SemaphoreType.DMA((NBUF,)),
            pltpu.SemaphoreType.DMA((NBUF,)),
        ],
    )
    def gather_kernel(idx_hbm, tab_hbm, out_hbm, idx_v, rows, gsem, wsem):
        wid = lax.axis_index("s") * NC + lax.axis_index("c")
        base = wid * per_w
        pltpu.sync_copy(idx_hbm.at[wid], idx_v)

        def fire_gather(g, s):
            pltpu.async_copy(tab_hbm.at[idx_v.at[g]], rows.at[s], gsem.at[s])

        def wait_gather(g, s):
            pltpu.make_async_copy(
                tab_hbm.at[idx_v.at[g]], rows.at[s], gsem.at[s]).wait()

        def fire_write(g, s):
            pltpu.async_copy(
                rows.at[s], out_hbm.at[pl.ds(base + g * CHUNK, CHUNK)],
                wsem.at[s])

        def wait_write(g, s):
            pltpu.make_async_copy(
                rows.at[s], out_hbm.at[pl.ds(base + g * CHUNK, CHUNK)],
                wsem.at[s]).wait()

        for b in range(NBUF):
            fire_gather(b, b)

        def body(g, carry):
            s = lax.rem(g, NBUF)
            wait_gather(g, s)
            fire_write(g, s)
            g2 = g + NBUF

            @pl.when(g2 < n_chunks)
            def _():
                wait_write(g, s)
                fire_gather(g2, s)

            return carry

        lax.fori_loop(0, n_chunks, body, 0)

        for b in range(NBUF):
            g = n_chunks - NBUF + b
            wait_write(g, g % NBUF)

    out = gather_kernel(idx3, embeddings)
    return out.reshape(B, S, D)
